# baseline (speedup 1.0000x reference)
import functools
import numpy as np

import jax
import jax.numpy as jnp
from jax import lax
from jax.experimental import pallas as pl
from jax.experimental.pallas import tpu as pltpu

_RATIO = 0.5
_DEPTH = 3


def _pad_to(x, m, axis=0):
    n = x.shape[axis]
    p = (-n) % m
    if p == 0:
        return x
    pads = [(0, 0)] * x.ndim
    pads[axis] = (0, p)
    return jnp.pad(x, pads)


def _mm_kernel(f_ref, w_ref, b_ref, o_ref, *, relu):
    acc = jnp.dot(f_ref[...], w_ref[...], preferred_element_type=jnp.float32)
    acc = acc + b_ref[...]
    if relu:
        acc = jnp.maximum(acc, 0.0)
    o_ref[...] = acc


def _mm(f, W, b, relu):
    n, K = f.shape
    H = W.shape[1]
    BN = 512
    fp = _pad_to(_pad_to(f, BN, 0), 128, 1)
    Wp = _pad_to(W, 128, 0)
    npad, Kp = fp.shape
    grid = (npad // BN,)
    out = pl.pallas_call(
        functools.partial(_mm_kernel, relu=relu),
        grid=grid,
        in_specs=[
            pl.BlockSpec((BN, Kp), lambda i: (i, 0)),
            pl.BlockSpec((Kp, H), lambda i: (0, 0)),
            pl.BlockSpec((1, H), lambda i: (0, 0)),
        ],
        out_specs=pl.BlockSpec((BN, H), lambda i: (i, 0)),
        out_shape=jax.ShapeDtypeStruct((npad, H), jnp.float32),
    )(fp, Wp, b.reshape(1, H))
    return out[:n]


def _mesh_conv(x, nbr, W, b, relu):
    xa = x[nbr[:, 0]]
    xb = x[nbr[:, 1]]
    xc = x[nbr[:, 2]]
    xd = x[nbr[:, 3]]
    f = jnp.concatenate(
        [x, jnp.abs(xa - xc), xa + xc, jnp.abs(xb - xd), xb + xd], axis=1)
    return _mm(f, W, b, relu)


def _pool(x, nbr, p):
    n = x.shape[0]
    score = (x @ p) / (jnp.linalg.norm(p) + 1e-12)
    k = int(np.ceil(_RATIO * n))
    vals, perm = jax.lax.top_k(score, k)
    xp = x[perm] * jnp.tanh(vals)[:, None]
    inv = jnp.full((n,), -1, dtype=jnp.int32).at[perm].set(
        jnp.arange(k, dtype=jnp.int32))
    nb = inv[nbr[perm]]
    selfi = jnp.arange(k, dtype=jnp.int32)[:, None]
    nbp = jnp.where(nb < 0, selfi, nb)
    return xp, nbp, perm


def kernel(x, edge_index, W_in, b_in, W_d1, b_d1, p1, W_d2, b_d2, p2,
           W_d3, b_d3, p3, W_u1, b_u1, W_u2, b_u2, W_u3, b_u3):
    nbr0 = edge_index[1].reshape(-1, 4).astype(jnp.int32)
    x = _mesh_conv(x, nbr0, W_in, b_in, True)
    down = [(W_d1, b_d1, p1), (W_d2, b_d2, p2), (W_d3, b_d3, p3)]
    up = [(W_u1, b_u1), (W_u2, b_u2), (W_u3, b_u3)]
    skips, perms, res_nbrs, sizes = [], [], [nbr0], []
    nbr = nbr0
    for (W, b, p) in down:
        skips.append(x)
        sizes.append(x.shape[0])
        x, nbr, perm = _pool(x, nbr, p)
        perms.append(perm)
        res_nbrs.append(nbr)
        x = _mesh_conv(x, nbr, W, b, True)
    for j in range(_DEPTH):
        i = _DEPTH - 1 - j
        W, b = up[j]
        u = jnp.zeros((sizes[i], x.shape[1]), x.dtype).at[perms[i]].set(x)
        x = u + skips[i]
        x = _mesh_conv(x, res_nbrs[i], W, b, j < _DEPTH - 1)
    return x
